# baseline (speedup 1.0000x reference)
import functools

import jax
import jax.numpy as jnp
from jax import lax
from jax.experimental import pallas as pl
from jax.experimental.pallas import tpu as pltpu
from jax.experimental.pallas import tpu_sc as plsc


_ROWS = 512


def _loss_body(logits_ref, targets_ref, loss_ref):
    t = targets_ref[0]
    x0 = logits_ref[0, 0]
    nclass = logits_ref.shape[1]
    m = x0
    for ci in range(1, nclass):
        m = jnp.maximum(m, logits_ref[0, ci])
    s = jnp.zeros_like(m)
    xt = jnp.zeros_like(m)
    for ci in range(nclass):
        xc = logits_ref[0, ci]
        s = s + jnp.exp(xc - m)
        xt = xt + jnp.where(t == ci, xc, 0.0)
    loss_ref[0] = (m - xt) + jnp.log(s)


def _compute_loss(logits, targets):
    n, c, h, w = logits.shape
    grid = (n, h // _ROWS)
    return pl.pallas_call(
        _loss_body,
        grid=grid,
        in_specs=[
            pl.BlockSpec((1, c, _ROWS, w), lambda b, r: (b, 0, r, 0)),
            pl.BlockSpec((1, _ROWS, w), lambda b, r: (b, r, 0)),
        ],
        out_specs=pl.BlockSpec((1, _ROWS, w), lambda b, r: (b, r, 0)),
        out_shape=jax.ShapeDtypeStruct((n, h, w), jnp.float32),
    )(logits, targets)


_N = 4 * 512 * 512
_NT = 16
_CH = _N // _NT
_NV = _CH // 16
_L1 = 2048
_L2 = 1024
_MIN_KEPT = 100000
_UNR = 16
_PLW = ((16 * (_L1 + 1) + 127) // 128) * 128
_PL2W = ((16 * (_L2 + 1) + 127) // 128) * 128


def _zero_fill(ref, nwords, dtype=jnp.int32):
    zv = jnp.zeros((16,), dtype)

    def body(i, carry):
        for u in range(_UNR):
            ref[pl.ds((i * _UNR + u) * 16, 16)] = zv
        return carry

    lax.fori_loop(0, nwords // (16 * _UNR), body, 0)


def _iota_fill(ref, nwords):
    base = lax.iota(jnp.int32, 16)

    def body(i, carry):
        for u in range(_UNR):
            ref[pl.ds((i * _UNR + u) * 16, 16)] = base + (i * _UNR + u) * 16
        return carry

    lax.fori_loop(0, nwords // (16 * _UNR), body, 0)


def _plane_reduce_merge(planes_v, lhist_v, iota_ref, hist_s, nbins):
    def body(j, carry):
        acc = planes_v[pl.ds(j * 16, 16)]
        for p in range(1, 16):
            acc = acc + planes_v[pl.ds(p * (nbins + 1) + j * 16, 16)]
        lhist_v[pl.ds(j * 16, 16)] = acc
        return carry

    lax.fori_loop(0, nbins // 16, body, 0)
    pltpu.sync_copy(lhist_v.at[pl.ds(0, nbins)], hist_s.at[iota_ref], add=True)


def _scan_desc(hist_v, nbins, k_rem, val_base=None):
    lane = lax.iota(jnp.int32, 16)

    def body(jj, carry):
        acc, facc, bfound, kfound, cfound, ffound = carry
        j = nbins // 16 - 1 - jj
        v = hist_v[pl.ds(j * 16, 16)]
        rv = lax.rev(v, (0,))
        cs = plsc.cumsum(rv)
        suf = acc + cs
        m = suf >= k_rem
        istar = plsc.all_reduce_ffs(m)
        istar_s = jnp.max(istar)
        hit = jnp.logical_and(istar_s < 16, bfound < 0)
        bchunk = j * 16 + 15 - istar_s
        ii = jnp.minimum(istar, 15)
        isel = lane == ii
        d = cs - rv
        d_s = jnp.sum(jnp.where(isel, d, 0))
        c_s = jnp.sum(jnp.where(isel, cs, 0))
        if val_base is None:
            f_s = jnp.float32(0.0)
            ftot = facc
        else:
            bins_rev = (val_base + j * 16 + 15) - lane
            vals = lax.bitcast_convert_type(bins_rev, jnp.float32)
            cf = plsc.cumsum(vals * rv.astype(jnp.float32))
            f_s = jnp.sum(jnp.where(isel, cf, 0.0))
            ftot = facc + jnp.max(cf)
        bfound = jnp.where(hit, bchunk, bfound)
        kfound = jnp.where(hit, k_rem - (acc + d_s), kfound)
        cfound = jnp.where(hit, acc + c_s, cfound)
        ffound = jnp.where(hit, facc + f_s, ffound)
        total = acc + jnp.max(cs)
        return total, ftot, bfound, kfound, cfound, ffound

    _, _, b, k_next, cnt_ge, fsum_ge = lax.fori_loop(
        0, nbins // 16, body,
        (jnp.int32(0), jnp.float32(0.0), jnp.int32(-1), jnp.int32(0),
         jnp.int32(0), jnp.float32(0.0)))
    return b, k_next, cnt_ge, fsum_ge


def _sc_body(loss_hbm, out_hbm, data_v, planes_v, lhist_v,
             iota2k_v, iota1k_v, bc_v, rowf_v, rowi_v, out_v, bcf_v, acc_v,
             hist1_s, hist23_s, zc_s, bc_s, pa_s, pb_s, pc_s):
    wid = lax.axis_index("s")
    lane = lax.iota(jnp.int32, 16)
    ones_i = jnp.ones((16,), jnp.int32)
    zf = jnp.zeros((16,), jnp.float32)

    _sc0 = jax.named_scope("p0_stage"); _sc0.__enter__()
    pltpu.sync_copy(
        loss_hbm.at[lax.shift_right_logical(wid, 2),
                    pl.ds((wid & 3) * (_CH // 512), _CH // 512)], data_v)
    _iota_fill(iota2k_v, _L1)
    _iota_fill(iota1k_v, _L2)
    _zero_fill(lhist_v, _L1)

    @pl.when(wid == 0)
    def _():
        pltpu.sync_copy(lhist_v, hist1_s)
        pltpu.sync_copy(lhist_v.at[pl.ds(0, _L2)], hist23_s)

    _zero_fill(planes_v, _PLW)
    plsc.subcore_barrier()
    _sc0.__exit__(None, None, None)
    _sc1 = jax.named_scope("p1_hist"); _sc1.__enter__()
    laneoff1 = lane * (_L1 + 1)

    def l1_body(i, carry):
        zc, ts = carry
        x = data_v[lax.shift_right_logical(i, 5), pl.ds((i & 31) * 16, 16)]
        bits = lax.bitcast_convert_type(x, jnp.int32)
        bkt = lax.shift_right_logical(bits, 20)
        plsc.addupdate_scatter(planes_v, [laneoff1 + bkt], ones_i)
        return zc + jnp.where(bits == 0, 1, 0), ts + x

    zc, ts = plsc.parallel_loop(
        0, _NV, unroll=_UNR,
        carry=(jnp.zeros((16,), jnp.int32), zf))(l1_body)
    acc_v[pl.ds(0, 16)] = ts
    _plane_reduce_merge(planes_v, lhist_v, iota2k_v, hist1_s, _L1)
    bc_v[pl.ds(0, 16)] = jnp.broadcast_to(jnp.sum(zc), (16,))
    pltpu.sync_copy(bc_v.at[pl.ds(0, 16)], zc_s.at[pl.ds(wid * 16, 16)])
    _zero_fill(planes_v, _PL2W)
    plsc.subcore_barrier()
    _sc1.__exit__(None, None, None)
    _sc2 = jax.named_scope("p2_scan1"); _sc2.__enter__()
    @pl.when(wid == 0)
    def _():
        pltpu.sync_copy(zc_s, rowi_v)
        tot = rowi_v[pl.ds(0, 16)]
        for r in range(1, 16):
            tot = tot + rowi_v[pl.ds(r * 16, 16)]
        cnt = _N - jnp.max(tot)
        k = jnp.minimum(jnp.maximum(_MIN_KEPT, (cnt * 7) // 10), cnt)
        pltpu.sync_copy(hist1_s, lhist_v)
        b1, k1, _, _ = _scan_desc(lhist_v, _L1, k)
        bc_v[pl.ds(0, 16)] = jnp.broadcast_to(b1, (16,))
        bc_v[pl.ds(16, 16)] = jnp.broadcast_to(k1, (16,))
        bc_v[pl.ds(32, 16)] = jnp.broadcast_to(cnt, (16,))
        pltpu.sync_copy(bc_v, bc_s)

    plsc.subcore_barrier()
    _sc2.__exit__(None, None, None)
    _sc3 = jax.named_scope("p3_hist2"); _sc3.__enter__()
    pltpu.sync_copy(bc_s, bc_v)
    b1v = bc_v[pl.ds(0, 16)]
    laneoff2 = lane * (_L2 + 1)

    def l2_body(i, carry):
        sa, ca = carry
        x = data_v[lax.shift_right_logical(i, 5), pl.ds((i & 31) * 16, 16)]
        bits = lax.bitcast_convert_type(x, jnp.int32)
        hi = lax.shift_right_logical(bits, 20)
        bkt = lax.shift_right_logical(bits, 10) & (_L2 - 1)
        plsc.addupdate_scatter(planes_v, [laneoff2 + bkt], ones_i,
                               mask=hi == b1v)
        above = hi > b1v
        return (sa + jnp.where(above, x, 0.0),
                ca + jnp.where(above, 1.0, 0.0))

    sa, ca = plsc.parallel_loop(0, _NV, unroll=_UNR, carry=(zf, zf))(l2_body)
    _plane_reduce_merge(planes_v, lhist_v, iota1k_v, hist23_s, _L2)
    _zero_fill(planes_v, _PL2W)
    plsc.subcore_barrier()
    _sc3.__exit__(None, None, None)
    _sc4 = jax.named_scope("p4_scan2"); _sc4.__enter__()
    @pl.when(wid == 0)
    def _():
        pltpu.sync_copy(hist23_s, lhist_v.at[pl.ds(0, _L2)])
        k1 = jnp.max(bc_v[pl.ds(16, 16)])
        b1 = jnp.max(bc_v[pl.ds(0, 16)])
        cntv = bc_v[pl.ds(32, 16)]
        b2, k2, _, _ = _scan_desc(lhist_v, _L2, k1)
        bc_v[pl.ds(0, 16)] = jnp.broadcast_to(b1 * _L2 + b2, (16,))
        bc_v[pl.ds(16, 16)] = jnp.broadcast_to(k2, (16,))
        bc_v[pl.ds(32, 16)] = cntv
        pltpu.sync_copy(bc_v, bc_s)
        _zero_fill(lhist_v, _L2)
        pltpu.sync_copy(lhist_v.at[pl.ds(0, _L2)], hist23_s)

    plsc.subcore_barrier()
    _sc4.__exit__(None, None, None)
    _sc5 = jax.named_scope("p5_hist3"); _sc5.__enter__()
    pltpu.sync_copy(bc_s, bc_v)
    p2v = bc_v[pl.ds(0, 16)]
    b1v3 = lax.shift_right_logical(p2v, 10)

    def l3_body(i, carry):
        s2, c2 = carry
        x = data_v[lax.shift_right_logical(i, 5), pl.ds((i & 31) * 16, 16)]
        bits = lax.bitcast_convert_type(x, jnp.int32)
        mid = lax.shift_right_logical(bits, 10)
        bkt = bits & (_L2 - 1)
        plsc.addupdate_scatter(planes_v, [laneoff2 + bkt], ones_i,
                               mask=mid == p2v)
        above = jnp.logical_and(mid > p2v,
                                lax.shift_right_logical(bits, 20) == b1v3)
        return (s2 + jnp.where(above, x, 0.0),
                c2 + jnp.where(above, 1.0, 0.0))

    s2, c2 = plsc.parallel_loop(0, _NV, unroll=_UNR, carry=(zf, zf))(l3_body)
    _plane_reduce_merge(planes_v, lhist_v, iota1k_v, hist23_s, _L2)

    ts = acc_v[pl.ds(0, 16)]
    bcf_v[pl.ds(0, 16)] = jnp.broadcast_to(jnp.sum(sa + s2), (16,))
    pltpu.sync_copy(bcf_v, pa_s.at[pl.ds(wid * 16, 16)])
    bcf_v[pl.ds(0, 16)] = jnp.broadcast_to(jnp.sum(ca + c2), (16,))
    pltpu.sync_copy(bcf_v, pb_s.at[pl.ds(wid * 16, 16)])
    bcf_v[pl.ds(0, 16)] = jnp.broadcast_to(jnp.sum(ts), (16,))
    pltpu.sync_copy(bcf_v, pc_s.at[pl.ds(wid * 16, 16)])
    plsc.subcore_barrier()
    _sc5.__exit__(None, None, None)
    _sc6 = jax.named_scope("p6_final"); _sc6.__enter__()
    @pl.when(wid == 0)
    def _():
        pltpu.sync_copy(hist23_s, lhist_v.at[pl.ds(0, _L2)])
        k2 = jnp.max(bc_v[pl.ds(16, 16)])
        p2 = jnp.max(bc_v[pl.ds(0, 16)])
        _, _, cnt_ge, fsum_ge = _scan_desc(lhist_v, _L2, k2,
                                           val_base=p2 * _L2)

        def row_total(src_s):
            pltpu.sync_copy(src_s, rowf_v)
            tot = rowf_v[pl.ds(0, 16)]
            for r in range(1, 16):
                tot = tot + rowf_v[pl.ds(r * 16, 16)]
            return tot

        out_v[pl.ds(0, 16)] = row_total(pa_s) + fsum_ge
        out_v[pl.ds(16, 16)] = row_total(pb_s) + cnt_ge.astype(jnp.float32)
        out_v[pl.ds(32, 16)] = bc_v[pl.ds(32, 16)].astype(jnp.float32)
        out_v[pl.ds(48, 16)] = row_total(pc_s)
        pltpu.sync_copy(out_v, out_hbm)

    _sc6.__exit__(None, None, None)


@jax.jit
def _ohem_select(loss_flat):
    mesh = plsc.VectorSubcoreMesh(
        core_axis_name="c", subcore_axis_name="s", num_cores=1)
    f = pl.kernel(
        _sc_body,
        out_type=jax.ShapeDtypeStruct((64,), jnp.float32),
        mesh=mesh,
        compiler_params=pltpu.CompilerParams(needs_layout_passes=False),
        scratch_types=[
            pltpu.VMEM((_CH // 512, 512), jnp.float32),
            pltpu.VMEM((_PLW,), jnp.int32),
            pltpu.VMEM((_L1,), jnp.int32),
            pltpu.VMEM((_L1,), jnp.int32),
            pltpu.VMEM((_L2,), jnp.int32),
            pltpu.VMEM((48,), jnp.int32),
            pltpu.VMEM((256,), jnp.float32),
            pltpu.VMEM((256,), jnp.int32),
            pltpu.VMEM((64,), jnp.float32),
            pltpu.VMEM((16,), jnp.float32),
            pltpu.VMEM((16,), jnp.float32),
            pltpu.VMEM_SHARED((_L1,), jnp.int32),
            pltpu.VMEM_SHARED((_L2,), jnp.int32),
            pltpu.VMEM_SHARED((256,), jnp.int32),
            pltpu.VMEM_SHARED((48,), jnp.int32),
            pltpu.VMEM_SHARED((256,), jnp.float32),
            pltpu.VMEM_SHARED((256,), jnp.float32),
            pltpu.VMEM_SHARED((256,), jnp.float32),
        ],
    )
    return f(loss_flat)


def kernel(logits, targets):
    loss = _compute_loss(logits, targets)
    o = _ohem_select(loss)
    ks, kc, cntf, ts = o[0], o[16], o[32], o[48]
    total = jnp.float32(loss.size)
    return jnp.where(cntf == 0.0, ts / total, ks / kc)


# speedup vs baseline: 1.0103x; 1.0103x over previous
import functools

import jax
import jax.numpy as jnp
from jax import lax
from jax.experimental import pallas as pl
from jax.experimental.pallas import tpu as pltpu
from jax.experimental.pallas import tpu_sc as plsc


_ROWS = 256


def _loss_body(logits_ref, targets_ref, loss_ref):
    t = targets_ref[0]
    x0 = logits_ref[0, 0]
    nclass = logits_ref.shape[1]
    m = x0
    for ci in range(1, nclass):
        m = jnp.maximum(m, logits_ref[0, ci])
    s = jnp.zeros_like(m)
    xt = jnp.zeros_like(m)
    for ci in range(nclass):
        xc = logits_ref[0, ci]
        s = s + jnp.exp(xc - m)
        xt = xt + jnp.where(t == ci, xc, 0.0)
    loss_ref[0] = (m - xt) + jnp.log(s)


def _compute_loss(logits, targets):
    n, c, h, w = logits.shape
    grid = (n, h // _ROWS)
    return pl.pallas_call(
        _loss_body,
        grid=grid,
        in_specs=[
            pl.BlockSpec((1, c, _ROWS, w), lambda b, r: (b, 0, r, 0)),
            pl.BlockSpec((1, _ROWS, w), lambda b, r: (b, r, 0)),
        ],
        out_specs=pl.BlockSpec((1, _ROWS, w), lambda b, r: (b, r, 0)),
        out_shape=jax.ShapeDtypeStruct((n, h, w), jnp.float32),
    )(logits, targets)


_N = 4 * 512 * 512
_NT = 16
_CH = _N // _NT
_NV = _CH // 16
_L1 = 2048
_L2 = 1024
_MIN_KEPT = 100000
_UNR = 16
_PLW = ((16 * (_L1 + 1) + 127) // 128) * 128
_PL2W = ((16 * (_L2 + 1) + 127) // 128) * 128


def _zero_fill(ref, nwords, dtype=jnp.int32):
    zv = jnp.zeros((16,), dtype)

    def body(i, carry):
        for u in range(_UNR):
            ref[pl.ds((i * _UNR + u) * 16, 16)] = zv
        return carry

    lax.fori_loop(0, nwords // (16 * _UNR), body, 0)


def _iota_fill(ref, nwords):
    base = lax.iota(jnp.int32, 16)

    def body(i, carry):
        for u in range(_UNR):
            ref[pl.ds((i * _UNR + u) * 16, 16)] = base + (i * _UNR + u) * 16
        return carry

    lax.fori_loop(0, nwords // (16 * _UNR), body, 0)


def _plane_reduce_merge(planes_v, lhist_v, iota_ref, hist_s, nbins):
    def body(j, carry):
        acc = planes_v[pl.ds(j * 16, 16)]
        for p in range(1, 16):
            acc = acc + planes_v[pl.ds(p * (nbins + 1) + j * 16, 16)]
        lhist_v[pl.ds(j * 16, 16)] = acc
        return carry

    lax.fori_loop(0, nbins // 16, body, 0)
    pltpu.sync_copy(lhist_v.at[pl.ds(0, nbins)], hist_s.at[iota_ref], add=True)


def _scan_desc(hist_v, nbins, k_rem, val_base=None):
    lane = lax.iota(jnp.int32, 16)

    def body(jj, carry):
        acc, facc, bfound, kfound, cfound, ffound = carry
        j = nbins // 16 - 1 - jj
        v = hist_v[pl.ds(j * 16, 16)]
        rv = lax.rev(v, (0,))
        cs = plsc.cumsum(rv)
        suf = acc + cs
        m = suf >= k_rem
        istar = plsc.all_reduce_ffs(m)
        istar_s = jnp.max(istar)
        hit = jnp.logical_and(istar_s < 16, bfound < 0)
        bchunk = j * 16 + 15 - istar_s
        ii = jnp.minimum(istar, 15)
        isel = lane == ii
        d = cs - rv
        d_s = jnp.sum(jnp.where(isel, d, 0))
        c_s = jnp.sum(jnp.where(isel, cs, 0))
        if val_base is None:
            f_s = jnp.float32(0.0)
            ftot = facc
        else:
            bins_rev = (val_base + j * 16 + 15) - lane
            vals = lax.bitcast_convert_type(bins_rev, jnp.float32)
            cf = plsc.cumsum(vals * rv.astype(jnp.float32))
            f_s = jnp.sum(jnp.where(isel, cf, 0.0))
            ftot = facc + jnp.max(cf)
        bfound = jnp.where(hit, bchunk, bfound)
        kfound = jnp.where(hit, k_rem - (acc + d_s), kfound)
        cfound = jnp.where(hit, acc + c_s, cfound)
        ffound = jnp.where(hit, facc + f_s, ffound)
        total = acc + jnp.max(cs)
        return total, ftot, bfound, kfound, cfound, ffound

    _, _, b, k_next, cnt_ge, fsum_ge = lax.fori_loop(
        0, nbins // 16, body,
        (jnp.int32(0), jnp.float32(0.0), jnp.int32(-1), jnp.int32(0),
         jnp.int32(0), jnp.float32(0.0)))
    return b, k_next, cnt_ge, fsum_ge


def _sc_body(loss_hbm, out_hbm, data_v, planes_v, lhist_v,
             iota2k_v, iota1k_v, bc_v, rowf_v, rowi_v, out_v, bcf_v, acc_v,
             hist1_s, hist23_s, zc_s, bc_s, pa_s, pb_s, pc_s):
    wid = lax.axis_index("s")
    lane = lax.iota(jnp.int32, 16)
    ones_i = jnp.ones((16,), jnp.int32)
    zf = jnp.zeros((16,), jnp.float32)

    _sc0 = jax.named_scope("p0_stage"); _sc0.__enter__()
    pltpu.sync_copy(
        loss_hbm.at[lax.shift_right_logical(wid, 2),
                    pl.ds((wid & 3) * (_CH // 512), _CH // 512)], data_v)
    _iota_fill(iota2k_v, _L1)
    _iota_fill(iota1k_v, _L2)
    _zero_fill(lhist_v, _L1)

    @pl.when(wid == 0)
    def _():
        pltpu.sync_copy(lhist_v, hist1_s)
        pltpu.sync_copy(lhist_v.at[pl.ds(0, _L2)], hist23_s)

    _zero_fill(planes_v, _PLW)
    plsc.subcore_barrier()
    _sc0.__exit__(None, None, None)
    _sc1 = jax.named_scope("p1_hist"); _sc1.__enter__()
    laneoff1 = lane * (_L1 + 1)

    def l1_body(i, carry):
        zc, ts = carry
        x = data_v[lax.shift_right_logical(i, 5), pl.ds((i & 31) * 16, 16)]
        bits = lax.bitcast_convert_type(x, jnp.int32)
        bkt = lax.shift_right_logical(bits, 20)
        plsc.addupdate_scatter(planes_v, [laneoff1 + bkt], ones_i)
        return zc + jnp.where(bits == 0, 1, 0), ts + x

    zc, ts = plsc.parallel_loop(
        0, _NV, unroll=_UNR,
        carry=(jnp.zeros((16,), jnp.int32), zf))(l1_body)
    acc_v[pl.ds(0, 16)] = ts
    _plane_reduce_merge(planes_v, lhist_v, iota2k_v, hist1_s, _L1)
    bc_v[pl.ds(0, 16)] = jnp.broadcast_to(jnp.sum(zc), (16,))
    pltpu.sync_copy(bc_v.at[pl.ds(0, 16)], zc_s.at[pl.ds(wid * 16, 16)])
    _zero_fill(planes_v, _PL2W)
    plsc.subcore_barrier()
    _sc1.__exit__(None, None, None)
    _sc2 = jax.named_scope("p2_scan1"); _sc2.__enter__()
    @pl.when(wid == 0)
    def _():
        pltpu.sync_copy(zc_s, rowi_v)
        tot = rowi_v[pl.ds(0, 16)]
        for r in range(1, 16):
            tot = tot + rowi_v[pl.ds(r * 16, 16)]
        cnt = _N - jnp.max(tot)
        k = jnp.minimum(jnp.maximum(_MIN_KEPT, (cnt * 7) // 10), cnt)
        pltpu.sync_copy(hist1_s, lhist_v)
        b1, k1, _, _ = _scan_desc(lhist_v, _L1, k)
        bc_v[pl.ds(0, 16)] = jnp.broadcast_to(b1, (16,))
        bc_v[pl.ds(16, 16)] = jnp.broadcast_to(k1, (16,))
        bc_v[pl.ds(32, 16)] = jnp.broadcast_to(cnt, (16,))
        pltpu.sync_copy(bc_v, bc_s)

    plsc.subcore_barrier()
    _sc2.__exit__(None, None, None)
    _sc3 = jax.named_scope("p3_hist2"); _sc3.__enter__()
    pltpu.sync_copy(bc_s, bc_v)
    b1v = bc_v[pl.ds(0, 16)]
    laneoff2 = lane * (_L2 + 1)

    def l2_body(i, carry):
        sa, ca = carry
        x = data_v[lax.shift_right_logical(i, 5), pl.ds((i & 31) * 16, 16)]
        bits = lax.bitcast_convert_type(x, jnp.int32)
        hi = lax.shift_right_logical(bits, 20)
        bkt = lax.shift_right_logical(bits, 10) & (_L2 - 1)
        plsc.addupdate_scatter(planes_v, [laneoff2 + bkt], ones_i,
                               mask=hi == b1v)
        above = hi > b1v
        return (sa + jnp.where(above, x, 0.0),
                ca + jnp.where(above, 1.0, 0.0))

    sa, ca = plsc.parallel_loop(0, _NV, unroll=_UNR, carry=(zf, zf))(l2_body)
    _plane_reduce_merge(planes_v, lhist_v, iota1k_v, hist23_s, _L2)
    _zero_fill(planes_v, _PL2W)
    plsc.subcore_barrier()
    _sc3.__exit__(None, None, None)
    _sc4 = jax.named_scope("p4_scan2"); _sc4.__enter__()
    @pl.when(wid == 0)
    def _():
        pltpu.sync_copy(hist23_s, lhist_v.at[pl.ds(0, _L2)])
        k1 = jnp.max(bc_v[pl.ds(16, 16)])
        b1 = jnp.max(bc_v[pl.ds(0, 16)])
        cntv = bc_v[pl.ds(32, 16)]
        b2, k2, _, _ = _scan_desc(lhist_v, _L2, k1)
        bc_v[pl.ds(0, 16)] = jnp.broadcast_to(b1 * _L2 + b2, (16,))
        bc_v[pl.ds(16, 16)] = jnp.broadcast_to(k2, (16,))
        bc_v[pl.ds(32, 16)] = cntv
        pltpu.sync_copy(bc_v, bc_s)
        _zero_fill(lhist_v, _L2)
        pltpu.sync_copy(lhist_v.at[pl.ds(0, _L2)], hist23_s)

    plsc.subcore_barrier()
    _sc4.__exit__(None, None, None)
    _sc5 = jax.named_scope("p5_hist3"); _sc5.__enter__()
    pltpu.sync_copy(bc_s, bc_v)
    p2v = bc_v[pl.ds(0, 16)]
    b1v3 = lax.shift_right_logical(p2v, 10)

    def l3_body(i, carry):
        s2, c2 = carry
        x = data_v[lax.shift_right_logical(i, 5), pl.ds((i & 31) * 16, 16)]
        bits = lax.bitcast_convert_type(x, jnp.int32)
        mid = lax.shift_right_logical(bits, 10)
        bkt = bits & (_L2 - 1)
        plsc.addupdate_scatter(planes_v, [laneoff2 + bkt], ones_i,
                               mask=mid == p2v)
        above = jnp.logical_and(mid > p2v,
                                lax.shift_right_logical(bits, 20) == b1v3)
        return (s2 + jnp.where(above, x, 0.0),
                c2 + jnp.where(above, 1.0, 0.0))

    s2, c2 = plsc.parallel_loop(0, _NV, unroll=_UNR, carry=(zf, zf))(l3_body)
    _plane_reduce_merge(planes_v, lhist_v, iota1k_v, hist23_s, _L2)

    ts = acc_v[pl.ds(0, 16)]
    bcf_v[pl.ds(0, 16)] = jnp.broadcast_to(jnp.sum(sa + s2), (16,))
    pltpu.sync_copy(bcf_v, pa_s.at[pl.ds(wid * 16, 16)])
    bcf_v[pl.ds(0, 16)] = jnp.broadcast_to(jnp.sum(ca + c2), (16,))
    pltpu.sync_copy(bcf_v, pb_s.at[pl.ds(wid * 16, 16)])
    bcf_v[pl.ds(0, 16)] = jnp.broadcast_to(jnp.sum(ts), (16,))
    pltpu.sync_copy(bcf_v, pc_s.at[pl.ds(wid * 16, 16)])
    plsc.subcore_barrier()
    _sc5.__exit__(None, None, None)
    _sc6 = jax.named_scope("p6_final"); _sc6.__enter__()
    @pl.when(wid == 0)
    def _():
        pltpu.sync_copy(hist23_s, lhist_v.at[pl.ds(0, _L2)])
        k2 = jnp.max(bc_v[pl.ds(16, 16)])
        p2 = jnp.max(bc_v[pl.ds(0, 16)])
        _, _, cnt_ge, fsum_ge = _scan_desc(lhist_v, _L2, k2,
                                           val_base=p2 * _L2)

        def row_total(src_s):
            pltpu.sync_copy(src_s, rowf_v)
            tot = rowf_v[pl.ds(0, 16)]
            for r in range(1, 16):
                tot = tot + rowf_v[pl.ds(r * 16, 16)]
            return tot

        out_v[pl.ds(0, 16)] = row_total(pa_s) + fsum_ge
        out_v[pl.ds(16, 16)] = row_total(pb_s) + cnt_ge.astype(jnp.float32)
        out_v[pl.ds(32, 16)] = bc_v[pl.ds(32, 16)].astype(jnp.float32)
        out_v[pl.ds(48, 16)] = row_total(pc_s)
        pltpu.sync_copy(out_v, out_hbm)

    _sc6.__exit__(None, None, None)


@jax.jit
def _ohem_select(loss_flat):
    mesh = plsc.VectorSubcoreMesh(
        core_axis_name="c", subcore_axis_name="s", num_cores=1)
    f = pl.kernel(
        _sc_body,
        out_type=jax.ShapeDtypeStruct((64,), jnp.float32),
        mesh=mesh,
        compiler_params=pltpu.CompilerParams(needs_layout_passes=False),
        scratch_types=[
            pltpu.VMEM((_CH // 512, 512), jnp.float32),
            pltpu.VMEM((_PLW,), jnp.int32),
            pltpu.VMEM((_L1,), jnp.int32),
            pltpu.VMEM((_L1,), jnp.int32),
            pltpu.VMEM((_L2,), jnp.int32),
            pltpu.VMEM((48,), jnp.int32),
            pltpu.VMEM((256,), jnp.float32),
            pltpu.VMEM((256,), jnp.int32),
            pltpu.VMEM((64,), jnp.float32),
            pltpu.VMEM((16,), jnp.float32),
            pltpu.VMEM((16,), jnp.float32),
            pltpu.VMEM_SHARED((_L1,), jnp.int32),
            pltpu.VMEM_SHARED((_L2,), jnp.int32),
            pltpu.VMEM_SHARED((256,), jnp.int32),
            pltpu.VMEM_SHARED((48,), jnp.int32),
            pltpu.VMEM_SHARED((256,), jnp.float32),
            pltpu.VMEM_SHARED((256,), jnp.float32),
            pltpu.VMEM_SHARED((256,), jnp.float32),
        ],
    )
    return f(loss_flat)


def kernel(logits, targets):
    loss = _compute_loss(logits, targets)
    o = _ohem_select(loss)
    ks, kc, cntf, ts = o[0], o[16], o[32], o[48]
    total = jnp.float32(loss.size)
    return jnp.where(cntf == 0.0, ts / total, ks / kc)


# speedup vs baseline: 1.1128x; 1.1015x over previous
import functools

import jax
import jax.numpy as jnp
from jax import lax
from jax.experimental import pallas as pl
from jax.experimental.pallas import tpu as pltpu
from jax.experimental.pallas import tpu_sc as plsc


_ROWS = 256


def _loss_body(logits_ref, targets_ref, loss_ref):
    t = targets_ref[0]
    x0 = logits_ref[0, 0]
    nclass = logits_ref.shape[1]
    m = x0
    for ci in range(1, nclass):
        m = jnp.maximum(m, logits_ref[0, ci])
    s = jnp.zeros_like(m)
    xt = jnp.zeros_like(m)
    for ci in range(nclass):
        xc = logits_ref[0, ci]
        s = s + jnp.exp(xc - m)
        xt = xt + jnp.where(t == ci, xc, 0.0)
    loss_ref[0] = (m - xt) + jnp.log(s)


def _compute_loss(logits, targets):
    n, c, h, w = logits.shape
    grid = (n, h // _ROWS)
    return pl.pallas_call(
        _loss_body,
        grid=grid,
        in_specs=[
            pl.BlockSpec((1, c, _ROWS, w), lambda b, r: (b, 0, r, 0)),
            pl.BlockSpec((1, _ROWS, w), lambda b, r: (b, r, 0)),
        ],
        out_specs=pl.BlockSpec((1, _ROWS, w), lambda b, r: (b, r, 0)),
        out_shape=jax.ShapeDtypeStruct((n, h, w), jnp.float32),
    )(logits, targets)


_N = 4 * 512 * 512
_NT = 16
_CH = _N // _NT
_NV = _CH // 16
_L1 = 2048
_L2 = 1024
_MIN_KEPT = 100000
_UNR = 8
_PLW = ((16 * (_L1 + 1) + 127) // 128) * 128
_PL2W = ((16 * (_L2 + 1) + 127) // 128) * 128


def _zero_fill(ref, nwords, dtype=jnp.int32):
    zv = jnp.zeros((16,), dtype)

    def body(i, carry):
        for u in range(_UNR):
            ref[pl.ds((i * _UNR + u) * 16, 16)] = zv
        return carry

    lax.fori_loop(0, nwords // (16 * _UNR), body, 0)


def _iota_fill(ref, nwords):
    base = lax.iota(jnp.int32, 16)

    def body(i, carry):
        for u in range(_UNR):
            ref[pl.ds((i * _UNR + u) * 16, 16)] = base + (i * _UNR + u) * 16
        return carry

    lax.fori_loop(0, nwords // (16 * _UNR), body, 0)


def _plane_reduce_merge(planes_v, lhist_v, iota_ref, hist_s, nbins):
    def body(j, carry):
        acc = planes_v[pl.ds(j * 16, 16)]
        for p in range(1, 16):
            acc = acc + planes_v[pl.ds(p * (nbins + 1) + j * 16, 16)]
        lhist_v[pl.ds(j * 16, 16)] = acc
        return carry

    lax.fori_loop(0, nbins // 16, body, 0)
    pltpu.sync_copy(lhist_v.at[pl.ds(0, nbins)], hist_s.at[iota_ref], add=True)


def _scan_desc(hist_v, nbins, k_rem, val_base=None):
    lane = lax.iota(jnp.int32, 16)

    def body(jj, carry):
        acc, facc, bfound, kfound, cfound, ffound = carry
        j = nbins // 16 - 1 - jj
        v = hist_v[pl.ds(j * 16, 16)]
        rv = lax.rev(v, (0,))
        cs = plsc.cumsum(rv)
        suf = acc + cs
        m = suf >= k_rem
        istar = plsc.all_reduce_ffs(m)
        istar_s = jnp.max(istar)
        hit = jnp.logical_and(istar_s < 16, bfound < 0)
        bchunk = j * 16 + 15 - istar_s
        ii = jnp.minimum(istar, 15)
        isel = lane == ii
        d = cs - rv
        d_s = jnp.sum(jnp.where(isel, d, 0))
        c_s = jnp.sum(jnp.where(isel, cs, 0))
        if val_base is None:
            f_s = jnp.float32(0.0)
            ftot = facc
        else:
            bins_rev = (val_base + j * 16 + 15) - lane
            vals = lax.bitcast_convert_type(bins_rev, jnp.float32)
            cf = plsc.cumsum(vals * rv.astype(jnp.float32))
            f_s = jnp.sum(jnp.where(isel, cf, 0.0))
            ftot = facc + jnp.max(cf)
        bfound = jnp.where(hit, bchunk, bfound)
        kfound = jnp.where(hit, k_rem - (acc + d_s), kfound)
        cfound = jnp.where(hit, acc + c_s, cfound)
        ffound = jnp.where(hit, facc + f_s, ffound)
        total = acc + jnp.max(cs)
        return total, ftot, bfound, kfound, cfound, ffound

    _, _, b, k_next, cnt_ge, fsum_ge = lax.fori_loop(
        0, nbins // 16, body,
        (jnp.int32(0), jnp.float32(0.0), jnp.int32(-1), jnp.int32(0),
         jnp.int32(0), jnp.float32(0.0)))
    return b, k_next, cnt_ge, fsum_ge


def _sc_body(loss_hbm, out_hbm, data_v, planes_v, lhist_v,
             iota2k_v, iota1k_v, bc_v, rowf_v, rowi_v, out_v, bcf_v, acc_v,
             hist1_s, hist23_s, zc_s, bc_s, pa_s, pb_s, pc_s):
    wid = lax.axis_index("s")
    lane = lax.iota(jnp.int32, 16)
    ones_i = jnp.ones((16,), jnp.int32)
    zf = jnp.zeros((16,), jnp.float32)

    pltpu.sync_copy(
        loss_hbm.at[lax.shift_right_logical(wid, 2),
                    pl.ds((wid & 3) * (_CH // 512), _CH // 512)], data_v)
    _iota_fill(iota2k_v, _L1)
    _iota_fill(iota1k_v, _L2)
    _zero_fill(lhist_v, _L1)

    @pl.when(wid == 0)
    def _():
        pltpu.sync_copy(lhist_v, hist1_s)
        pltpu.sync_copy(lhist_v.at[pl.ds(0, _L2)], hist23_s)

    _zero_fill(planes_v, _PLW)
    plsc.subcore_barrier()
    laneoff1 = lane * (_L1 + 1)

    def l1_body(i, carry):
        zc, ts = carry
        x = data_v[lax.shift_right_logical(i, 5), pl.ds((i & 31) * 16, 16)]
        bits = lax.bitcast_convert_type(x, jnp.int32)
        bkt = lax.shift_right_logical(bits, 20)
        plsc.addupdate_scatter(planes_v, [laneoff1 + bkt], ones_i)
        return zc + jnp.where(bits == 0, 1, 0), ts + x

    zc, ts = plsc.parallel_loop(
        0, _NV, unroll=_UNR,
        carry=(jnp.zeros((16,), jnp.int32), zf))(l1_body)
    acc_v[pl.ds(0, 16)] = ts
    _plane_reduce_merge(planes_v, lhist_v, iota2k_v, hist1_s, _L1)
    bc_v[pl.ds(0, 16)] = jnp.broadcast_to(jnp.sum(zc), (16,))
    pltpu.sync_copy(bc_v.at[pl.ds(0, 16)], zc_s.at[pl.ds(wid * 16, 16)])
    _zero_fill(planes_v, _PL2W)
    plsc.subcore_barrier()
    @pl.when(wid == 0)
    def _():
        pltpu.sync_copy(zc_s, rowi_v)
        tot = rowi_v[pl.ds(0, 16)]
        for r in range(1, 16):
            tot = tot + rowi_v[pl.ds(r * 16, 16)]
        cnt = _N - jnp.max(tot)
        k = jnp.minimum(jnp.maximum(_MIN_KEPT, (cnt * 7) // 10), cnt)
        pltpu.sync_copy(hist1_s, lhist_v)
        b1, k1, _, _ = _scan_desc(lhist_v, _L1, k)
        bc_v[pl.ds(0, 16)] = jnp.broadcast_to(b1, (16,))
        bc_v[pl.ds(16, 16)] = jnp.broadcast_to(k1, (16,))
        bc_v[pl.ds(32, 16)] = jnp.broadcast_to(cnt, (16,))
        pltpu.sync_copy(bc_v, bc_s)

    plsc.subcore_barrier()
    pltpu.sync_copy(bc_s, bc_v)
    b1v = bc_v[pl.ds(0, 16)]
    laneoff2 = lane * (_L2 + 1)

    def l2_body(i, carry):
        sa, ca = carry
        x = data_v[lax.shift_right_logical(i, 5), pl.ds((i & 31) * 16, 16)]
        bits = lax.bitcast_convert_type(x, jnp.int32)
        hi = lax.shift_right_logical(bits, 20)
        bkt = lax.shift_right_logical(bits, 10) & (_L2 - 1)
        plsc.addupdate_scatter(planes_v, [laneoff2 + bkt], ones_i,
                               mask=hi == b1v)
        above = hi > b1v
        return (sa + jnp.where(above, x, 0.0),
                ca + jnp.where(above, 1.0, 0.0))

    sa, ca = plsc.parallel_loop(0, _NV, unroll=_UNR, carry=(zf, zf))(l2_body)
    _plane_reduce_merge(planes_v, lhist_v, iota1k_v, hist23_s, _L2)
    _zero_fill(planes_v, _PL2W)
    plsc.subcore_barrier()
    @pl.when(wid == 0)
    def _():
        pltpu.sync_copy(hist23_s, lhist_v.at[pl.ds(0, _L2)])
        k1 = jnp.max(bc_v[pl.ds(16, 16)])
        b1 = jnp.max(bc_v[pl.ds(0, 16)])
        cntv = bc_v[pl.ds(32, 16)]
        b2, k2, _, _ = _scan_desc(lhist_v, _L2, k1)
        bc_v[pl.ds(0, 16)] = jnp.broadcast_to(b1 * _L2 + b2, (16,))
        bc_v[pl.ds(16, 16)] = jnp.broadcast_to(k2, (16,))
        bc_v[pl.ds(32, 16)] = cntv
        pltpu.sync_copy(bc_v, bc_s)
        _zero_fill(lhist_v, _L2)
        pltpu.sync_copy(lhist_v.at[pl.ds(0, _L2)], hist23_s)

    plsc.subcore_barrier()
    pltpu.sync_copy(bc_s, bc_v)
    p2v = bc_v[pl.ds(0, 16)]
    b1v3 = lax.shift_right_logical(p2v, 10)

    def l3_body(i, carry):
        s2, c2 = carry
        x = data_v[lax.shift_right_logical(i, 5), pl.ds((i & 31) * 16, 16)]
        bits = lax.bitcast_convert_type(x, jnp.int32)
        mid = lax.shift_right_logical(bits, 10)
        bkt = bits & (_L2 - 1)
        plsc.addupdate_scatter(planes_v, [laneoff2 + bkt], ones_i,
                               mask=mid == p2v)
        above = jnp.logical_and(mid > p2v,
                                lax.shift_right_logical(bits, 20) == b1v3)
        return (s2 + jnp.where(above, x, 0.0),
                c2 + jnp.where(above, 1.0, 0.0))

    s2, c2 = plsc.parallel_loop(0, _NV, unroll=_UNR, carry=(zf, zf))(l3_body)
    _plane_reduce_merge(planes_v, lhist_v, iota1k_v, hist23_s, _L2)

    ts = acc_v[pl.ds(0, 16)]
    bcf_v[pl.ds(0, 16)] = jnp.broadcast_to(jnp.sum(sa + s2), (16,))
    pltpu.sync_copy(bcf_v, pa_s.at[pl.ds(wid * 16, 16)])
    bcf_v[pl.ds(0, 16)] = jnp.broadcast_to(jnp.sum(ca + c2), (16,))
    pltpu.sync_copy(bcf_v, pb_s.at[pl.ds(wid * 16, 16)])
    bcf_v[pl.ds(0, 16)] = jnp.broadcast_to(jnp.sum(ts), (16,))
    pltpu.sync_copy(bcf_v, pc_s.at[pl.ds(wid * 16, 16)])
    plsc.subcore_barrier()
    @pl.when(wid == 0)
    def _():
        pltpu.sync_copy(hist23_s, lhist_v.at[pl.ds(0, _L2)])
        k2 = jnp.max(bc_v[pl.ds(16, 16)])
        p2 = jnp.max(bc_v[pl.ds(0, 16)])
        _, _, cnt_ge, fsum_ge = _scan_desc(lhist_v, _L2, k2,
                                           val_base=p2 * _L2)

        def row_total(src_s):
            pltpu.sync_copy(src_s, rowf_v)
            tot = rowf_v[pl.ds(0, 16)]
            for r in range(1, 16):
                tot = tot + rowf_v[pl.ds(r * 16, 16)]
            return tot

        out_v[pl.ds(0, 16)] = row_total(pa_s) + fsum_ge
        out_v[pl.ds(16, 16)] = row_total(pb_s) + cnt_ge.astype(jnp.float32)
        out_v[pl.ds(32, 16)] = bc_v[pl.ds(32, 16)].astype(jnp.float32)
        out_v[pl.ds(48, 16)] = row_total(pc_s)
        pltpu.sync_copy(out_v, out_hbm)


@jax.jit
def _ohem_select(loss_flat):
    mesh = plsc.VectorSubcoreMesh(
        core_axis_name="c", subcore_axis_name="s", num_cores=1)
    f = pl.kernel(
        _sc_body,
        out_type=jax.ShapeDtypeStruct((64,), jnp.float32),
        mesh=mesh,
        compiler_params=pltpu.CompilerParams(needs_layout_passes=False),
        scratch_types=[
            pltpu.VMEM((_CH // 512, 512), jnp.float32),
            pltpu.VMEM((_PLW,), jnp.int32),
            pltpu.VMEM((_L1,), jnp.int32),
            pltpu.VMEM((_L1,), jnp.int32),
            pltpu.VMEM((_L2,), jnp.int32),
            pltpu.VMEM((48,), jnp.int32),
            pltpu.VMEM((256,), jnp.float32),
            pltpu.VMEM((256,), jnp.int32),
            pltpu.VMEM((64,), jnp.float32),
            pltpu.VMEM((16,), jnp.float32),
            pltpu.VMEM((16,), jnp.float32),
            pltpu.VMEM_SHARED((_L1,), jnp.int32),
            pltpu.VMEM_SHARED((_L2,), jnp.int32),
            pltpu.VMEM_SHARED((256,), jnp.int32),
            pltpu.VMEM_SHARED((48,), jnp.int32),
            pltpu.VMEM_SHARED((256,), jnp.float32),
            pltpu.VMEM_SHARED((256,), jnp.float32),
            pltpu.VMEM_SHARED((256,), jnp.float32),
        ],
    )
    return f(loss_flat)


def kernel(logits, targets):
    loss = _compute_loss(logits, targets)
    o = _ohem_select(loss)
    ks, kc, cntf, ts = o[0], o[16], o[32], o[48]
    total = jnp.float32(loss.size)
    return jnp.where(cntf == 0.0, ts / total, ks / kc)
